# baseline (speedup 1.0000x reference)
import jax
import jax.numpy as jnp
from jax import lax
from jax.experimental import pallas as pl
from jax.experimental.pallas import tpu as pltpu
from jax.experimental.pallas import tpu_sc as plsc

_NC = 2
_NS = 16
_NW = _NC * _NS
_ROWS = 1_000_000
_CHUNK = 200
_NCHUNKS = _ROWS // _CHUNK
_NFULL = _NCHUNKS // _NW
_EXTRA = _NCHUNKS - _NFULL * _NW
_KMAX = _NFULL + 1
_D = 4
_AHEAD = _D // 2


def _sc_copy(in_hbm, out_hbm, *refs):
    bufs = refs[:_D]
    rsem = refs[_D:2 * _D]
    wsem = refs[2 * _D:3 * _D]
    wid = lax.axis_index("s") * _NC + lax.axis_index("c")
    cnt = _NFULL + jnp.where(wid < _EXTRA, 1, 0)
    base = (wid * _NFULL + jnp.minimum(wid, _EXTRA)) * _CHUNK

    def rd(k):
        b = k % _D
        return pltpu.make_async_copy(
            in_hbm.at[pl.ds(base + k * _CHUNK, _CHUNK), :], bufs[b], rsem[b])

    def wr(k):
        b = k % _D
        return pltpu.make_async_copy(
            bufs[b], out_hbm.at[pl.ds(base + k * _CHUNK, _CHUNK), :], wsem[b])

    def guarded(k, op):
        if k < _NFULL:
            op()
        else:
            pl.when(k < cnt)(op)

    for k in range(min(_AHEAD, _KMAX)):
        guarded(k, rd(k).start)
    for k in range(_KMAX):
        if k >= _AHEAD:
            guarded(k - _AHEAD, wr(k - _AHEAD).wait)
        if k + _AHEAD < _KMAX:
            guarded(k + _AHEAD, rd(k + _AHEAD).start)
        guarded(k, rd(k).wait)
        guarded(k, wr(k).start)
    for k in range(max(_KMAX - _AHEAD, 0), _KMAX):
        guarded(k, wr(k).wait)


def kernel(table):
    n, d = table.shape
    mesh = plsc.VectorSubcoreMesh(core_axis_name="c", subcore_axis_name="s")
    run = pl.kernel(
        _sc_copy,
        out_type=jax.ShapeDtypeStruct((n, d), table.dtype),
        mesh=mesh,
        scratch_types=(
            [pltpu.VMEM((_CHUNK, 32), jnp.float32) for _ in range(_D)]
            + [pltpu.SemaphoreType.DMA for _ in range(2 * _D)]
        ),
    )
    return run(table)
